# baseline (speedup 1.0000x reference)
import functools

import jax
import jax.numpy as jnp
from jax import lax
from jax.experimental import pallas as pl
from jax.experimental.pallas import tpu as pltpu
from jax.experimental.pallas import tpu_sc as plsc

N = 320000
C = 128
NJ = C // 16
BQ = 4
NVOX = 40 * 40 * 40
NSEG = NVOX * BQ
SUB = 200
VPB = SUB // BQ
NBLK = NSEG // SUB
RC = 128
ZACC = SUB * C
SCH = 2000

_info = plsc.get_sparse_core_info()
NC, NS, L = _info.num_cores, _info.num_subcores, _info.num_lanes
W = NC * NS
NBW = NBLK // W
RPT = N // NS
LCNW = 1296 * L
QSL = (NBW + 2) * L


def _seg_mean_body(feat_hbm, ranks_hbm, out_hbm,
                   fbuf0, fbuf1, rbuf0, rbuf1, acc0, acc1, cnt,
                   sbuf, rsfw, wsum, scntw,
                   sem0, sem1, osem0, osem1):
    fbufs = (fbuf0, fbuf1)
    rbufs = (rbuf0, rbuf1)
    sems = (sem0, sem1)
    accs = (acc0, acc1)
    osems = (osem0, osem1)
    sid = lax.axis_index("s")
    cid = lax.axis_index("c")
    w = sid * NC + cid

    zero16 = jnp.zeros((L,), jnp.float32)
    one16 = jnp.full((L,), 1.0, jnp.float32)
    zero16i = jnp.zeros((L,), jnp.int32)
    for a in accs:
        for j in range(NJ):
            a[pl.ds(ZACC + j * L, L)] = zero16

    def _blk(r):
        return lax.shift_right_logical(
            lax.shift_right_logical(r, 3) * 5243, 17)

    @plsc.parallel_loop(0, LCNW // L, unroll=8)
    def _wz(q):
        rsfw[pl.ds(q * L, L)] = zero16i

    g0 = sid * RPT
    pltpu.sync_copy(
        ranks_hbm.at[pl.ds(
            pl.multiple_of(jnp.maximum(g0 - 8, 0), 8), 8)],
        sbuf.at[pl.ds(0, 8)])
    prev0 = jnp.where(sid == 0, jnp.int32(-1), _blk(sbuf[pl.ds(0, L)][7]))

    def scan_chunk(ch, pv):
        a = pl.multiple_of(g0 + ch * SCH, 8)
        pltpu.sync_copy(ranks_hbm.at[pl.ds(a, SCH)], sbuf)

        def vec_body(v, pvv):
            rv = sbuf[pl.ds(v * L, L)]
            lastb = _blk(rv[15])

            @pl.when(lastb != pvv)
            def _():
                pb = pvv
                for l in range(L):
                    bl = _blk(rv[l])
                    row = a + v * L + l

                    def wq(q, _):
                        rsfw[pl.ds(q * L, L)] = jnp.full((L,), row,
                                                         jnp.int32)
                        return 0

                    lax.fori_loop(pb + 1, bl + 1, wq, 0)
                    pb = bl

            return lastb

        return lax.fori_loop(0, SCH // L, vec_body, pv)

    pv_final = lax.fori_loop(0, RPT // SCH, scan_chunk, prev0)

    @pl.when(sid == NS - 1)
    def _():
        def wq(q, _):
            rsfw[pl.ds(q * L, L)] = jnp.full((L,), N, jnp.int32)
            return 0

        lax.fori_loop(pv_final + 1, NBLK + 1, wq, 0)

    pltpu.sync_copy(
        rsfw,
        scntw.at[pl.ds(pl.multiple_of((cid * NS + sid) * LCNW, 8), LCNW)])
    plsc.subcore_barrier()

    qbase = w * NBW * L
    pltpu.sync_copy(
        scntw.at[pl.ds(pl.multiple_of(cid * NS * LCNW + qbase, 8), QSL)],
        wsum)
    for g in range(1, NS):
        pltpu.sync_copy(
            scntw.at[pl.ds(
                pl.multiple_of((cid * NS + g) * LCNW + qbase, 8), QSL)],
            sbuf.at[pl.ds(0, QSL)])

        @plsc.parallel_loop(0, QSL // L, unroll=4)
        def _m(q):
            wsum[pl.ds(q * L, L)] = (wsum[pl.ds(q * L, L)]
                                     + sbuf[pl.ds(q * L, L)])

    def _issue(k, bi, base):
        cstart = pl.multiple_of(jnp.minimum(base + k * RC, N - RC), 8)
        pltpu.async_copy(
            feat_hbm.at[pl.ds(pl.multiple_of(cstart * C, 128), RC * C)],
            fbufs[bi], sems[bi])
        pltpu.async_copy(ranks_hbm.at[pl.ds(cstart, RC)],
                         rbufs[bi].at[pl.ds(0, RC)], sems[bi])

    def _wait_out(pb):
        for _ in range(BQ):
            pltpu.make_async_copy(
                accs[pb].at[pl.ds(0, VPB * C)],
                out_hbm.at[pl.ds(0, VPB * C)], osems[pb]).wait()

    rs00 = wsum[pl.ds(0, L)][0]
    rs01 = wsum[pl.ds(L, L)][0]

    @pl.when(rs01 > rs00)
    def _():
        _issue(0, 0, lax.bitwise_and(rs00, jnp.int32(-8)))

    def block_body(t, acc, osem, pb):
        s_lo = (w * NBW + t) * SUB
        vlo = s_lo // BQ
        rs = wsum[pl.ds(t * L, L)][0]
        re = wsum[pl.ds((t + 1) * L, L)][0]
        base = lax.bitwise_and(rs, jnp.int32(-8))
        nchunks = jnp.where(re > rs, (re - base + (RC - 1)) // RC, 0)

        @pl.when(t >= 2)
        def _():
            _wait_out(pb)

        @plsc.parallel_loop(0, SUB, unroll=8)
        def _czero(p):
            cnt[pl.ds(p * L, L)] = zero16
            for j in range(NJ):
                acc[pl.ds(p * C + j * L, L)] = zero16

        def pair_body(kk, carry):
            for bi in range(2):
                k = kk * 2 + bi
                fbuf = fbufs[bi]
                rbuf = rbufs[bi]

                @pl.when(k + 1 < nchunks)
                def _():
                    _issue(k + 1, 1 - bi, base)

                @pl.when(k < nchunks)
                def _():
                    pltpu.make_async_copy(
                        feat_hbm.at[pl.ds(0, RC * C)], fbuf,
                        sems[bi]).wait()
                    pltpu.make_async_copy(
                        ranks_hbm.at[pl.ds(0, RC)],
                        rbuf.at[pl.ds(0, RC)], sems[bi]).wait()

                nom = base + k * RC
                cstart = pl.multiple_of(jnp.minimum(nom, N - RC), 8)
                i_lo = jnp.maximum(rs, nom) - cstart
                i_hi = jnp.minimum(re, nom + RC) - cstart

                @plsc.parallel_loop(i_lo, jnp.maximum(i_lo, i_hi),
                                    unroll=8)
                def _rows(i):
                    seg = rbuf[pl.ds(i, L)][0]
                    rel = seg - s_lo
                    b2 = lax.bitwise_and(rel, 3)
                    vl = lax.shift_right_logical(rel, 2)
                    p = b2 * VPB + vl
                    off = p * C
                    fb = i * C
                    for j in range(NJ):
                        plsc.addupdate(acc.at[pl.ds(off + j * L, L)],
                                       fbuf[pl.ds(fb + j * L, L)])
                    plsc.addupdate(cnt.at[pl.ds(p * L, L)], one16)
            return carry

        lax.fori_loop(0, (nchunks + 1) // 2, pair_body, 0)

        rs_n0 = wsum[pl.ds((t + 1) * L, L)][0]
        rs_n1 = wsum[pl.ds((t + 2) * L, L)][0]

        @pl.when((t + 1 < NBW) & (rs_n1 > rs_n0))
        def _():
            _issue(0, 0, lax.bitwise_and(rs_n0, jnp.int32(-8)))

        @plsc.parallel_loop(0, SUB, unroll=8)
        def _div(p):
            c_v = cnt[pl.ds(p * L, L)]
            c_s = c_v[0]
            off = p * C
            src = jnp.where(c_s > 0.0, off, ZACC)
            inv_v = one16 / jnp.maximum(c_v, one16)
            for j in range(NJ):
                a = acc[pl.ds(src + j * L, L)]
                acc[pl.ds(off + j * L, L)] = a * inv_v

        for b in range(BQ):
            pltpu.async_copy(
                acc.at[pl.ds(b * VPB * C, VPB * C)],
                out_hbm.at[pl.ds(
                    pl.multiple_of((b * NVOX + vlo) * C, 128), VPB * C)],
                osem)

    def pairblock_body(tt, _):
        for pb in range(2):
            block_body(tt * 2 + pb, accs[pb], osems[pb], pb)
        return 0

    lax.fori_loop(0, NBW // 2, pairblock_body, 0)
    for pb in range(2):
        _wait_out(pb)


_seg_mean = functools.partial(
    pl.kernel,
    mesh=plsc.VectorSubcoreMesh(core_axis_name="c", subcore_axis_name="s"),
    out_type=jax.ShapeDtypeStruct((BQ * NVOX * C,), jnp.float32),
    scratch_types=[
        pltpu.VMEM((RC * C,), jnp.float32),
        pltpu.VMEM((RC * C,), jnp.float32),
        pltpu.VMEM((RC + L,), jnp.int32),
        pltpu.VMEM((RC + L,), jnp.int32),
        pltpu.VMEM((SUB * C + C,), jnp.float32),
        pltpu.VMEM((SUB * C + C,), jnp.float32),
        pltpu.VMEM((SUB * L,), jnp.float32),
        pltpu.VMEM((SCH,), jnp.int32),
        pltpu.VMEM((LCNW,), jnp.int32),
        pltpu.VMEM((QSL,), jnp.int32),
        pltpu.HBM((W * LCNW,), jnp.int32),
        pltpu.SemaphoreType.DMA,
        pltpu.SemaphoreType.DMA,
        pltpu.SemaphoreType.DMA,
        pltpu.SemaphoreType.DMA,
    ],
)(_seg_mean_body)


def kernel(features, ranks):
    ranks_i32 = ranks.astype(jnp.int32)
    out = _seg_mean(features.reshape(-1), ranks_i32)
    return out.reshape(BQ, 40, 40, 40, C)


# speedup vs baseline: 1.0287x; 1.0287x over previous
import functools

import jax
import jax.numpy as jnp
from jax import lax
from jax.experimental import pallas as pl
from jax.experimental.pallas import tpu as pltpu
from jax.experimental.pallas import tpu_sc as plsc

N = 320000
C = 128
NJ = C // 16
BQ = 4
NVOX = 40 * 40 * 40
NSEG = NVOX * BQ
SUB = 200
VPB = SUB // BQ
NBLK = NSEG // SUB
RC = 128
ZACC = SUB * C
SCH = 2000

_info = plsc.get_sparse_core_info()
NC, NS, L = _info.num_cores, _info.num_subcores, _info.num_lanes
W = NC * NS
NBW = NBLK // W
RPT = N // NS
LCNW = 1296 * L
QSL = (NBW + 2) * L


def _seg_mean_body(feat_hbm, ranks_hbm, out_hbm,
                   fbuf0, fbuf1, rbuf0, rbuf1, acc0, acc1, cnt,
                   sbuf, rsfw, wsum, scntw,
                   sem0, sem1, osem0, osem1):
    fbufs = (fbuf0, fbuf1)
    rbufs = (rbuf0, rbuf1)
    sems = (sem0, sem1)
    accs = (acc0, acc1)
    osems = (osem0, osem1)
    sid = lax.axis_index("s")
    cid = lax.axis_index("c")
    w = sid * NC + cid

    zero16 = jnp.zeros((L,), jnp.float32)
    one16 = jnp.full((L,), 1.0, jnp.float32)
    zero16i = jnp.zeros((L,), jnp.int32)
    for a in accs:
        for j in range(NJ):
            a[pl.ds(ZACC + j * L, L)] = zero16

    def _blk(r):
        return lax.shift_right_logical(
            lax.shift_right_logical(r, 3) * 5243, 17)

    @plsc.parallel_loop(0, LCNW // L, unroll=8)
    def _wz(q):
        rsfw[pl.ds(q * L, L)] = zero16i

    g0 = sid * RPT
    pltpu.sync_copy(
        ranks_hbm.at[pl.ds(
            pl.multiple_of(jnp.maximum(g0 - 8, 0), 8), 8)],
        sbuf.at[pl.ds(0, 8)])
    prev0 = jnp.where(sid == 0, jnp.int32(-1), _blk(sbuf[pl.ds(0, L)][7]))

    def scan_chunk(ch, pv):
        a = pl.multiple_of(g0 + ch * SCH, 8)
        pltpu.sync_copy(ranks_hbm.at[pl.ds(a, SCH)], sbuf)

        def vec_body(v, pvv):
            rv = sbuf[pl.ds(v * L, L)]
            lastb = _blk(rv[15])

            @pl.when(lastb != pvv)
            def _():
                pb = pvv
                for l in range(L):
                    bl = _blk(rv[l])
                    row = a + v * L + l

                    def wq(q, _):
                        rsfw[pl.ds(q * L, L)] = jnp.full((L,), row,
                                                         jnp.int32)
                        return 0

                    lax.fori_loop(pb + 1, bl + 1, wq, 0)
                    pb = bl

            return lastb

        return lax.fori_loop(0, SCH // L, vec_body, pv)

    pv_final = lax.fori_loop(0, RPT // SCH, scan_chunk, prev0)

    @pl.when(sid == NS - 1)
    def _():
        def wq(q, _):
            rsfw[pl.ds(q * L, L)] = jnp.full((L,), N, jnp.int32)
            return 0

        lax.fori_loop(pv_final + 1, NBLK + 1, wq, 0)

    pltpu.sync_copy(
        rsfw,
        scntw.at[pl.ds(pl.multiple_of((cid * NS + sid) * LCNW, 8), LCNW)])
    plsc.subcore_barrier()

    qbase = w * NBW * L
    pltpu.sync_copy(
        scntw.at[pl.ds(pl.multiple_of(cid * NS * LCNW + qbase, 8), QSL)],
        wsum)
    for g in range(1, NS):
        pltpu.sync_copy(
            scntw.at[pl.ds(
                pl.multiple_of((cid * NS + g) * LCNW + qbase, 8), QSL)],
            sbuf.at[pl.ds(0, QSL)])

        @plsc.parallel_loop(0, QSL // L, unroll=4)
        def _m(q):
            wsum[pl.ds(q * L, L)] = (wsum[pl.ds(q * L, L)]
                                     + sbuf[pl.ds(q * L, L)])

    def _issue(k, bi, base):
        cstart = pl.multiple_of(jnp.minimum(base + k * RC, N - RC), 8)
        pltpu.async_copy(
            feat_hbm.at[pl.ds(pl.multiple_of(cstart * C, 128), RC * C)],
            fbufs[bi], sems[bi])
        pltpu.async_copy(ranks_hbm.at[pl.ds(cstart, RC)],
                         rbufs[bi].at[pl.ds(0, RC)], sems[bi])

    def _wait_out(pb):
        for _ in range(BQ):
            pltpu.make_async_copy(
                accs[pb].at[pl.ds(0, VPB * C)],
                out_hbm.at[pl.ds(0, VPB * C)], osems[pb]).wait()

    rs00 = wsum[pl.ds(0, L)][0]
    rs01 = wsum[pl.ds(L, L)][0]

    @pl.when(rs01 > rs00)
    def _():
        _issue(0, 0, lax.bitwise_and(rs00, jnp.int32(-8)))

    def block_body(t, acc, osem, pb):
        s_lo = (w * NBW + t) * SUB
        vlo = s_lo // BQ
        rs = wsum[pl.ds(t * L, L)][0]
        re = wsum[pl.ds((t + 1) * L, L)][0]
        base = lax.bitwise_and(rs, jnp.int32(-8))
        nchunks = jnp.where(re > rs, (re - base + (RC - 1)) // RC, 0)

        @pl.when(t >= 2)
        def _():
            _wait_out(pb)

        @plsc.parallel_loop(0, SUB, unroll=8)
        def _czero(p):
            cnt[pl.ds(p * L, L)] = zero16
            for j in range(NJ):
                acc[pl.ds(p * C + j * L, L)] = zero16

        def pair_body(kk, carry):
            for bi in range(2):
                k = kk * 2 + bi
                fbuf = fbufs[bi]
                rbuf = rbufs[bi]

                @pl.when(k + 1 < nchunks)
                def _():
                    _issue(k + 1, 1 - bi, base)

                @pl.when(k < nchunks)
                def _():
                    pltpu.make_async_copy(
                        feat_hbm.at[pl.ds(0, RC * C)], fbuf,
                        sems[bi]).wait()
                    pltpu.make_async_copy(
                        ranks_hbm.at[pl.ds(0, RC)],
                        rbuf.at[pl.ds(0, RC)], sems[bi]).wait()

                nom = base + k * RC
                cstart = pl.multiple_of(jnp.minimum(nom, N - RC), 8)
                i_lo = jnp.maximum(rs, nom) - cstart
                i_hi = jnp.minimum(re, nom + RC) - cstart

                @plsc.parallel_loop(i_lo, jnp.maximum(i_lo, i_hi),
                                    unroll=4)
                def _rows(i):
                    seg = rbuf[pl.ds(i, L)][0]
                    rel = seg - s_lo
                    b2 = lax.bitwise_and(rel, 3)
                    vl = lax.shift_right_logical(rel, 2)
                    p = b2 * VPB + vl
                    off = p * C
                    fb = i * C
                    for j in range(NJ):
                        plsc.addupdate(acc.at[pl.ds(off + j * L, L)],
                                       fbuf[pl.ds(fb + j * L, L)])
                    plsc.addupdate(cnt.at[pl.ds(p * L, L)], one16)
            return carry

        lax.fori_loop(0, (nchunks + 1) // 2, pair_body, 0)

        rs_n0 = wsum[pl.ds((t + 1) * L, L)][0]
        rs_n1 = wsum[pl.ds((t + 2) * L, L)][0]

        @pl.when((t + 1 < NBW) & (rs_n1 > rs_n0))
        def _():
            _issue(0, 0, lax.bitwise_and(rs_n0, jnp.int32(-8)))

        @plsc.parallel_loop(0, SUB, unroll=8)
        def _div(p):
            c_v = cnt[pl.ds(p * L, L)]
            c_s = c_v[0]
            off = p * C
            src = jnp.where(c_s > 0.0, off, ZACC)
            inv_v = one16 / jnp.maximum(c_v, one16)
            for j in range(NJ):
                a = acc[pl.ds(src + j * L, L)]
                acc[pl.ds(off + j * L, L)] = a * inv_v

        for b in range(BQ):
            pltpu.async_copy(
                acc.at[pl.ds(b * VPB * C, VPB * C)],
                out_hbm.at[pl.ds(
                    pl.multiple_of((b * NVOX + vlo) * C, 128), VPB * C)],
                osem)

    def pairblock_body(tt, _):
        for pb in range(2):
            block_body(tt * 2 + pb, accs[pb], osems[pb], pb)
        return 0

    lax.fori_loop(0, NBW // 2, pairblock_body, 0)
    for pb in range(2):
        _wait_out(pb)


_seg_mean = functools.partial(
    pl.kernel,
    mesh=plsc.VectorSubcoreMesh(core_axis_name="c", subcore_axis_name="s"),
    out_type=jax.ShapeDtypeStruct((BQ * NVOX * C,), jnp.float32),
    scratch_types=[
        pltpu.VMEM((RC * C,), jnp.float32),
        pltpu.VMEM((RC * C,), jnp.float32),
        pltpu.VMEM((RC + L,), jnp.int32),
        pltpu.VMEM((RC + L,), jnp.int32),
        pltpu.VMEM((SUB * C + C,), jnp.float32),
        pltpu.VMEM((SUB * C + C,), jnp.float32),
        pltpu.VMEM((SUB * L,), jnp.float32),
        pltpu.VMEM((SCH,), jnp.int32),
        pltpu.VMEM((LCNW,), jnp.int32),
        pltpu.VMEM((QSL,), jnp.int32),
        pltpu.HBM((W * LCNW,), jnp.int32),
        pltpu.SemaphoreType.DMA,
        pltpu.SemaphoreType.DMA,
        pltpu.SemaphoreType.DMA,
        pltpu.SemaphoreType.DMA,
    ],
)(_seg_mean_body)


def kernel(features, ranks):
    ranks_i32 = ranks.astype(jnp.int32)
    out = _seg_mean(features.reshape(-1), ranks_i32)
    return out.reshape(BQ, 40, 40, 40, C)


# speedup vs baseline: 1.0335x; 1.0047x over previous
import functools

import jax
import jax.numpy as jnp
from jax import lax
from jax.experimental import pallas as pl
from jax.experimental.pallas import tpu as pltpu
from jax.experimental.pallas import tpu_sc as plsc

N = 320000
C = 128
NJ = C // 16
BQ = 4
NVOX = 40 * 40 * 40
NSEG = NVOX * BQ
SUB = 200
VPB = SUB // BQ
NBLK = NSEG // SUB
RC = 128
ZACC = SUB * C
SCH = 2000

_info = plsc.get_sparse_core_info()
NC, NS, L = _info.num_cores, _info.num_subcores, _info.num_lanes
W = NC * NS
NBW = NBLK // W
RPT = N // NS
LCNW = 1296 * L
QSL = (NBW + 2) * L


def _seg_mean_body(feat_hbm, ranks_hbm, out_hbm,
                   fbuf0, fbuf1, rbuf0, rbuf1, acc0, acc1, cnt,
                   sbuf, rsfw, wsum, scntw,
                   sem0, sem1, osem0, osem1):
    fbufs = (fbuf0, fbuf1)
    rbufs = (rbuf0, rbuf1)
    sems = (sem0, sem1)
    accs = (acc0, acc1)
    osems = (osem0, osem1)
    sid = lax.axis_index("s")
    cid = lax.axis_index("c")
    w = sid * NC + cid

    zero16 = jnp.zeros((L,), jnp.float32)
    one16 = jnp.full((L,), 1.0, jnp.float32)
    zero16i = jnp.zeros((L,), jnp.int32)
    for a in accs:
        for j in range(NJ):
            a[pl.ds(ZACC + j * L, L)] = zero16

    def _blk(r):
        return lax.shift_right_logical(
            lax.shift_right_logical(r, 3) * 5243, 17)

    @plsc.parallel_loop(0, LCNW // L, unroll=8)
    def _wz(q):
        rsfw[pl.ds(q * L, L)] = zero16i

    g0 = sid * RPT
    pltpu.sync_copy(
        ranks_hbm.at[pl.ds(
            pl.multiple_of(jnp.maximum(g0 - 8, 0), 8), 8)],
        sbuf.at[pl.ds(0, 8)])
    prev0 = jnp.where(sid == 0, jnp.int32(-1), _blk(sbuf[pl.ds(0, L)][7]))

    def scan_chunk(ch, pv):
        a = pl.multiple_of(g0 + ch * SCH, 8)
        pltpu.sync_copy(ranks_hbm.at[pl.ds(a, SCH)], sbuf)

        def vec_body(v, pvv):
            rv = sbuf[pl.ds(v * L, L)]
            lastb = _blk(rv[15])

            @pl.when(lastb != pvv)
            def _():
                pb = pvv
                for l in range(L):
                    bl = _blk(rv[l])
                    row = a + v * L + l

                    def wq(q, _):
                        rsfw[pl.ds(q * L, L)] = jnp.full((L,), row,
                                                         jnp.int32)
                        return 0

                    lax.fori_loop(pb + 1, bl + 1, wq, 0)
                    pb = bl

            return lastb

        return lax.fori_loop(0, SCH // L, vec_body, pv)

    pv_final = lax.fori_loop(0, RPT // SCH, scan_chunk, prev0)

    @pl.when(sid == NS - 1)
    def _():
        def wq(q, _):
            rsfw[pl.ds(q * L, L)] = jnp.full((L,), N, jnp.int32)
            return 0

        lax.fori_loop(pv_final + 1, NBLK + 1, wq, 0)

    pltpu.sync_copy(
        rsfw,
        scntw.at[pl.ds(pl.multiple_of((cid * NS + sid) * LCNW, 8), LCNW)])
    plsc.subcore_barrier()

    qbase = w * NBW * L
    pltpu.sync_copy(
        scntw.at[pl.ds(pl.multiple_of(cid * NS * LCNW + qbase, 8), QSL)],
        wsum)
    for g in range(1, NS):
        pltpu.sync_copy(
            scntw.at[pl.ds(
                pl.multiple_of((cid * NS + g) * LCNW + qbase, 8), QSL)],
            sbuf.at[pl.ds(0, QSL)])

        @plsc.parallel_loop(0, QSL // L, unroll=4)
        def _m(q):
            wsum[pl.ds(q * L, L)] = (wsum[pl.ds(q * L, L)]
                                     + sbuf[pl.ds(q * L, L)])

    def _issue(k, bi, base):
        cstart = pl.multiple_of(jnp.minimum(base + k * RC, N - RC), 8)
        pltpu.async_copy(
            feat_hbm.at[pl.ds(pl.multiple_of(cstart * C, 128), RC * C)],
            fbufs[bi], sems[bi])
        pltpu.async_copy(ranks_hbm.at[pl.ds(cstart, RC)],
                         rbufs[bi].at[pl.ds(0, RC)], sems[bi])

    def _wait_out(pb):
        for _ in range(BQ):
            pltpu.make_async_copy(
                accs[pb].at[pl.ds(0, VPB * C)],
                out_hbm.at[pl.ds(0, VPB * C)], osems[pb]).wait()

    rs00 = wsum[pl.ds(0, L)][0]
    rs01 = wsum[pl.ds(L, L)][0]

    @pl.when(rs01 > rs00)
    def _():
        _issue(0, 0, lax.bitwise_and(rs00, jnp.int32(-8)))

    def block_body(t, acc, osem, pb):
        s_lo = (w * NBW + t) * SUB
        vlo = s_lo // BQ
        rs = wsum[pl.ds(t * L, L)][0]
        re = wsum[pl.ds((t + 1) * L, L)][0]
        base = lax.bitwise_and(rs, jnp.int32(-8))
        nchunks = jnp.where(re > rs, (re - base + (RC - 1)) // RC, 0)

        @pl.when(t >= 2)
        def _():
            _wait_out(pb)

        @plsc.parallel_loop(0, SUB, unroll=4)
        def _czero(p):
            cnt[pl.ds(p * L, L)] = zero16
            for j in range(NJ):
                acc[pl.ds(p * C + j * L, L)] = zero16

        def pair_body(kk, carry):
            for bi in range(2):
                k = kk * 2 + bi
                fbuf = fbufs[bi]
                rbuf = rbufs[bi]

                @pl.when(k + 1 < nchunks)
                def _():
                    _issue(k + 1, 1 - bi, base)

                @pl.when(k < nchunks)
                def _():
                    pltpu.make_async_copy(
                        feat_hbm.at[pl.ds(0, RC * C)], fbuf,
                        sems[bi]).wait()
                    pltpu.make_async_copy(
                        ranks_hbm.at[pl.ds(0, RC)],
                        rbuf.at[pl.ds(0, RC)], sems[bi]).wait()

                nom = base + k * RC
                cstart = pl.multiple_of(jnp.minimum(nom, N - RC), 8)
                i_lo = jnp.maximum(rs, nom) - cstart
                i_hi = jnp.minimum(re, nom + RC) - cstart

                @plsc.parallel_loop(i_lo, jnp.maximum(i_lo, i_hi),
                                    unroll=4)
                def _rows(i):
                    seg = rbuf[pl.ds(i, L)][0]
                    rel = seg - s_lo
                    b2 = lax.bitwise_and(rel, 3)
                    vl = lax.shift_right_logical(rel, 2)
                    p = b2 * VPB + vl
                    off = p * C
                    fb = i * C
                    for j in range(NJ):
                        plsc.addupdate(acc.at[pl.ds(off + j * L, L)],
                                       fbuf[pl.ds(fb + j * L, L)])
                    plsc.addupdate(cnt.at[pl.ds(p * L, L)], one16)
            return carry

        lax.fori_loop(0, (nchunks + 1) // 2, pair_body, 0)

        rs_n0 = wsum[pl.ds((t + 1) * L, L)][0]
        rs_n1 = wsum[pl.ds((t + 2) * L, L)][0]

        @pl.when((t + 1 < NBW) & (rs_n1 > rs_n0))
        def _():
            _issue(0, 0, lax.bitwise_and(rs_n0, jnp.int32(-8)))

        @plsc.parallel_loop(0, SUB, unroll=4)
        def _div(p):
            c_v = cnt[pl.ds(p * L, L)]
            c_s = c_v[0]
            off = p * C
            src = jnp.where(c_s > 0.0, off, ZACC)
            inv_v = one16 / jnp.maximum(c_v, one16)
            for j in range(NJ):
                a = acc[pl.ds(src + j * L, L)]
                acc[pl.ds(off + j * L, L)] = a * inv_v

        for b in range(BQ):
            pltpu.async_copy(
                acc.at[pl.ds(b * VPB * C, VPB * C)],
                out_hbm.at[pl.ds(
                    pl.multiple_of((b * NVOX + vlo) * C, 128), VPB * C)],
                osem)

    def pairblock_body(tt, _):
        for pb in range(2):
            block_body(tt * 2 + pb, accs[pb], osems[pb], pb)
        return 0

    lax.fori_loop(0, NBW // 2, pairblock_body, 0)
    for pb in range(2):
        _wait_out(pb)


_seg_mean = functools.partial(
    pl.kernel,
    mesh=plsc.VectorSubcoreMesh(core_axis_name="c", subcore_axis_name="s"),
    out_type=jax.ShapeDtypeStruct((BQ * NVOX * C,), jnp.float32),
    scratch_types=[
        pltpu.VMEM((RC * C,), jnp.float32),
        pltpu.VMEM((RC * C,), jnp.float32),
        pltpu.VMEM((RC + L,), jnp.int32),
        pltpu.VMEM((RC + L,), jnp.int32),
        pltpu.VMEM((SUB * C + C,), jnp.float32),
        pltpu.VMEM((SUB * C + C,), jnp.float32),
        pltpu.VMEM((SUB * L,), jnp.float32),
        pltpu.VMEM((SCH,), jnp.int32),
        pltpu.VMEM((LCNW,), jnp.int32),
        pltpu.VMEM((QSL,), jnp.int32),
        pltpu.HBM((W * LCNW,), jnp.int32),
        pltpu.SemaphoreType.DMA,
        pltpu.SemaphoreType.DMA,
        pltpu.SemaphoreType.DMA,
        pltpu.SemaphoreType.DMA,
    ],
)(_seg_mean_body)


def kernel(features, ranks):
    ranks_i32 = ranks.astype(jnp.int32)
    out = _seg_mean(features.reshape(-1), ranks_i32)
    return out.reshape(BQ, 40, 40, 40, C)
